# baseline (speedup 1.0000x reference)
import jax
import jax.numpy as jnp
from jax.experimental import pallas as pl
from jax.experimental.pallas import tpu as pltpu

N_NODES = 10000
N_EDGES = 160000
D_FEAT = 128
C00 = 0.28209479177387814
C1 = 0.4886025119029199

EDGE_BLK = 2000
NODE_BLK = 400


def _dense_body(x_ref, wc_ref, wsc_ref, hg_ref, short_ref):
    xb = x_ref[...]
    hg_ref[...] = jnp.dot(xb, wc_ref[...], preferred_element_type=jnp.float32)
    short_ref[...] = jnp.dot(xb, wsc_ref[...], preferred_element_type=jnp.float32)


def _edge_body(s_ref, r_ref, pos_ref, hg_ref, short_ref, out_ref):
    @pl.when(pl.program_id(0) == 0)
    def _init():
        out_ref[:, :D_FEAT] = short_ref[...]
        out_ref[:, D_FEAT:] = jnp.zeros((N_NODES, 192), jnp.float32)

    def body(e, carry):
        s = s_ref[0, 0, e]
        r = r_ref[0, 0, e]
        ps = pos_ref[pl.ds(s, 1), :]
        pr = pos_ref[pl.ds(r, 1), :]
        rel = pr - ps
        d = jnp.sum(rel * rel)
        inv = C1 / (jnp.sqrt(d) + 1e-12)
        shy = rel[0, 1] * inv
        shz = rel[0, 2] * inv
        shx = rel[0, 0] * inv
        hg = hg_ref[pl.ds(s, 1), :]
        h = hg[:, :D_FEAT]
        g = hg[:, D_FEAT:]
        msg = jnp.concatenate([h, g * shy, g * shz, g * shx], axis=1)
        out_ref[pl.ds(r, 1), :] += msg
        return carry

    jax.lax.fori_loop(0, EDGE_BLK, body, 0)


@jax.jit
def kernel(x, positions, senders, receivers, W0, W1, Wsc):
    s0 = 1.0 / (16.0 * jnp.sqrt(2.0 * D_FEAT))
    s1 = 1.0 / (16.0 * jnp.sqrt(1.0 * D_FEAT))
    ssc = 1.0 / jnp.sqrt(1.0 * D_FEAT)
    w0c = (W0[:D_FEAT] + C00 * W0[D_FEAT:]) * s0
    wc = jnp.concatenate([w0c, W1 * s1], axis=1)
    wsc = Wsc * ssc

    n_blocks = N_NODES // NODE_BLK
    hg, short = pl.pallas_call(
        _dense_body,
        grid=(n_blocks,),
        in_specs=[
            pl.BlockSpec((NODE_BLK, D_FEAT), lambda i: (i, 0)),
            pl.BlockSpec((D_FEAT, 192), lambda i: (0, 0)),
            pl.BlockSpec((D_FEAT, D_FEAT), lambda i: (0, 0)),
        ],
        out_specs=[
            pl.BlockSpec((NODE_BLK, 192), lambda i: (i, 0)),
            pl.BlockSpec((NODE_BLK, D_FEAT), lambda i: (i, 0)),
        ],
        out_shape=[
            jax.ShapeDtypeStruct((N_NODES, 192), jnp.float32),
            jax.ShapeDtypeStruct((N_NODES, D_FEAT), jnp.float32),
        ],
    )(x, wc, wsc)

    pos4 = jnp.pad(positions, ((0, 0), (0, 1)))
    e_blocks = N_EDGES // EDGE_BLK
    s3 = senders.reshape(e_blocks, 1, EDGE_BLK)
    r3 = receivers.reshape(e_blocks, 1, EDGE_BLK)

    acc = pl.pallas_call(
        _edge_body,
        grid=(e_blocks,),
        in_specs=[
            pl.BlockSpec((1, 1, EDGE_BLK), lambda i: (i, 0, 0),
                         memory_space=pltpu.SMEM),
            pl.BlockSpec((1, 1, EDGE_BLK), lambda i: (i, 0, 0),
                         memory_space=pltpu.SMEM),
            pl.BlockSpec((N_NODES, 4), lambda i: (0, 0)),
            pl.BlockSpec((N_NODES, 192), lambda i: (0, 0)),
            pl.BlockSpec((N_NODES, D_FEAT), lambda i: (0, 0)),
        ],
        out_specs=pl.BlockSpec((N_NODES, 320), lambda i: (0, 0)),
        out_shape=jax.ShapeDtypeStruct((N_NODES, 320), jnp.float32),
    )(s3, r3, pos4, hg, short)

    node_s = acc[:, :D_FEAT]
    v = acc[:, D_FEAT:].reshape(N_NODES, 3, 64).transpose(0, 2, 1)
    return jnp.concatenate([node_s, v.reshape(N_NODES, 192)], axis=1)


# speedup vs baseline: 1.0018x; 1.0018x over previous
import jax
import jax.numpy as jnp
from jax.experimental import pallas as pl
from jax.experimental.pallas import tpu as pltpu

N_NODES = 10000
N_EDGES = 160000
D_FEAT = 128
C00 = 0.28209479177387814
C1 = 0.4886025119029199

EDGE_BLK = 2000
NODE_BLK = 400


def _dense_body(x_ref, wc_ref, wsc_ref, hg_ref, short_ref):
    xb = x_ref[...]
    hg_ref[...] = jnp.dot(xb, wc_ref[...], preferred_element_type=jnp.float32)
    short_ref[...] = jnp.dot(xb, wsc_ref[...], preferred_element_type=jnp.float32)


def _edge_body(s_ref, r_ref, pos_ref, hg_ref, out_ref):
    @pl.when(pl.program_id(1) == 0)
    def _init():
        out_ref[...] = jnp.zeros((1, N_NODES, 320), jnp.float32)

    def body(e, carry):
        s = s_ref[0, 0, e]
        r = r_ref[0, 0, e]
        ps = pos_ref[pl.ds(s, 1), :]
        pr = pos_ref[pl.ds(r, 1), :]
        rel = pr - ps
        d = jnp.sum(rel * rel)
        inv = C1 / (jnp.sqrt(d) + 1e-12)
        shy = rel[0, 1] * inv
        shz = rel[0, 2] * inv
        shx = rel[0, 0] * inv
        hg = hg_ref[pl.ds(s, 1), :]
        h = hg[:, :D_FEAT]
        g = hg[:, D_FEAT:]
        msg = jnp.concatenate([h, g * shy, g * shz, g * shx], axis=1)
        out_ref[0, pl.ds(r, 1), :] += msg
        return carry

    jax.lax.fori_loop(0, EDGE_BLK, body, 0)


@jax.jit
def kernel(x, positions, senders, receivers, W0, W1, Wsc):
    s0 = 1.0 / (16.0 * jnp.sqrt(2.0 * D_FEAT))
    s1 = 1.0 / (16.0 * jnp.sqrt(1.0 * D_FEAT))
    ssc = 1.0 / jnp.sqrt(1.0 * D_FEAT)
    w0c = (W0[:D_FEAT] + C00 * W0[D_FEAT:]) * s0
    wc = jnp.concatenate([w0c, W1 * s1], axis=1)
    wsc = Wsc * ssc

    n_blocks = N_NODES // NODE_BLK
    hg, short = pl.pallas_call(
        _dense_body,
        grid=(n_blocks,),
        in_specs=[
            pl.BlockSpec((NODE_BLK, D_FEAT), lambda i: (i, 0)),
            pl.BlockSpec((D_FEAT, 192), lambda i: (0, 0)),
            pl.BlockSpec((D_FEAT, D_FEAT), lambda i: (0, 0)),
        ],
        out_specs=[
            pl.BlockSpec((NODE_BLK, 192), lambda i: (i, 0)),
            pl.BlockSpec((NODE_BLK, D_FEAT), lambda i: (i, 0)),
        ],
        out_shape=[
            jax.ShapeDtypeStruct((N_NODES, 192), jnp.float32),
            jax.ShapeDtypeStruct((N_NODES, D_FEAT), jnp.float32),
        ],
    )(x, wc, wsc)

    pos4 = jnp.pad(positions, ((0, 0), (0, 1)))
    e_blocks = N_EDGES // EDGE_BLK
    s3 = senders.reshape(e_blocks, 1, EDGE_BLK)
    r3 = receivers.reshape(e_blocks, 1, EDGE_BLK)

    n_par = 2
    acc2 = pl.pallas_call(
        _edge_body,
        grid=(n_par, e_blocks // n_par),
        in_specs=[
            pl.BlockSpec((1, 1, EDGE_BLK),
                         lambda p, i: (p * (N_EDGES // EDGE_BLK // 2) + i, 0, 0),
                         memory_space=pltpu.SMEM),
            pl.BlockSpec((1, 1, EDGE_BLK),
                         lambda p, i: (p * (N_EDGES // EDGE_BLK // 2) + i, 0, 0),
                         memory_space=pltpu.SMEM),
            pl.BlockSpec((N_NODES, 4), lambda p, i: (0, 0)),
            pl.BlockSpec((N_NODES, 192), lambda p, i: (0, 0)),
        ],
        out_specs=pl.BlockSpec((1, N_NODES, 320), lambda p, i: (p, 0, 0)),
        out_shape=jax.ShapeDtypeStruct((n_par, N_NODES, 320), jnp.float32),
        compiler_params=pltpu.CompilerParams(
            dimension_semantics=("parallel", "arbitrary")),
    )(s3, r3, pos4, hg)

    acc = acc2[0] + acc2[1]
    node_s = short + acc[:, :D_FEAT]
    v = acc[:, D_FEAT:].reshape(N_NODES, 3, 64).transpose(0, 2, 1)
    return jnp.concatenate([node_s, v.reshape(N_NODES, 192)], axis=1)
